# baseline (speedup 1.0000x reference)
import functools

import jax
import jax.numpy as jnp
from jax import lax
from jax.experimental import pallas as pl
from jax.experimental.pallas import tpu as pltpu
from jax.experimental.pallas import tpu_sc as plsc

EMBED_DIM = 32
BATCH = 16384

_NUM_CORES = 2
_NUM_SUBCORES = 16
_NUM_WORKERS = _NUM_CORES * _NUM_SUBCORES
_B_PER_W = BATCH // _NUM_WORKERS
_CHUNK = 128
_N_CHUNK = _B_PER_W // _CHUNK

_mesh = plsc.VectorSubcoreMesh(core_axis_name="c", subcore_axis_name="s")


@functools.partial(
    pl.kernel,
    mesh=_mesh,
    out_type=jax.ShapeDtypeStruct((BATCH // _CHUNK, _CHUNK, EMBED_DIM),
                                  jnp.float32),
    scratch_types=[
        pltpu.VMEM((_N_CHUNK, _CHUNK), jnp.int32),
        pltpu.VMEM((_N_CHUNK, _CHUNK, EMBED_DIM), jnp.float32),
        pltpu.SemaphoreType.DMA,
    ],
    compiler_params=pltpu.CompilerParams(use_tc_tiling_on_sc=False),
)
def _sc_gather(idx_hbm, table_hbm, out_hbm, idx_v, rows_v, sem):
    wid = lax.axis_index("s") * _NUM_CORES + lax.axis_index("c")
    row0 = wid * _N_CHUNK
    pltpu.sync_copy(idx_hbm.at[pl.ds(row0, _N_CHUNK)], idx_v)
    copies = [
        pltpu.async_copy(table_hbm.at[idx_v.at[j]], rows_v.at[j], sem)
        for j in range(_N_CHUNK)
    ]
    for cp in copies:
        cp.wait()
    pltpu.sync_copy(rows_v, out_hbm.at[pl.ds(row0, _N_CHUNK)])


def kernel(c, table):
    idx2d = c.astype(jnp.int32).reshape(BATCH // _CHUNK, _CHUNK)
    out = _sc_gather(idx2d, table)
    return out.reshape(BATCH, EMBED_DIM)


# speedup vs baseline: 3.8683x; 3.8683x over previous
import functools

import jax
import jax.numpy as jnp
from jax import lax
from jax.experimental import pallas as pl
from jax.experimental.pallas import tpu as pltpu
from jax.experimental.pallas import tpu_sc as plsc

EMBED_DIM = 32
BATCH = 16384
N_CLASSES = 1_000_000

_NUM_CORES = 2
_NUM_SUBCORES = 16
_NUM_WORKERS = _NUM_CORES * _NUM_SUBCORES
_B_PER_W = BATCH // _NUM_WORKERS
_SUB = 16
_N_SUB = _B_PER_W // _SUB
_LANES = 128

_mesh = plsc.VectorSubcoreMesh(core_axis_name="c", subcore_axis_name="s")


@functools.partial(
    pl.kernel,
    mesh=_mesh,
    out_type=jax.ShapeDtypeStruct((EMBED_DIM, BATCH), jnp.float32),
    scratch_types=[
        pltpu.VMEM((_B_PER_W,), jnp.int32),
        pltpu.VMEM((_SUB, 4, 8, _LANES), jnp.float32),
        pltpu.VMEM((EMBED_DIM, _LANES), jnp.float32),
        pltpu.SemaphoreType.DMA,
    ],
    compiler_params=pltpu.CompilerParams(needs_layout_passes=False),
)
def _sc_lane_gather(idx_hbm, t3_hbm, out_hbm, idx_v, gbuf, obuf, sem):
    wid = lax.axis_index("s") * _NUM_CORES + lax.axis_index("c")
    base = wid * _B_PER_W
    pltpu.sync_copy(idx_hbm.at[pl.ds(base, _B_PER_W)], idx_v)

    def stage(sub, carry):
        cvec = idx_v[pl.ds(sub * _SUB, _SUB)]
        for i in range(_SUB):
            c = cvec[i]
            blk = pl.multiple_of((c >> 7) << 7, _LANES)
            pltpu.async_copy(
                t3_hbm.at[:, :, pl.ds(blk, _LANES)], gbuf.at[i], sem
            )
        for i in range(_SUB):
            pltpu.make_async_copy(
                t3_hbm.at[:, :, pl.ds(0, _LANES)], gbuf.at[i], sem
            ).wait()

        pvec = lax.bitwise_and(cvec, _LANES - 1)
        ivec = lax.iota(jnp.int32, _SUB)
        col = lax.rem(sub, jnp.int32(8)) * _SUB
        for j in range(EMBED_DIM):
            a = jnp.full((_SUB,), j // 8, jnp.int32)
            b = jnp.full((_SUB,), j % 8, jnp.int32)
            vals = plsc.load_gather(gbuf, [ivec, a, b, pvec])
            obuf.at[j][pl.ds(col, _SUB)] = vals

        @pl.when(lax.rem(sub, jnp.int32(8)) == 7)
        def _():
            tile = (sub // 8) * _LANES
            off = pl.multiple_of(base + tile, _LANES)
            pltpu.sync_copy(obuf, out_hbm.at[:, pl.ds(off, _LANES)])

        return carry

    lax.fori_loop(0, _N_SUB, stage, 0)


def kernel(c, table):
    t3 = table.T.reshape(4, 8, N_CLASSES)
    out_t = _sc_lane_gather(c.astype(jnp.int32), t3)
    return out_t.T


# speedup vs baseline: 3.9097x; 1.0107x over previous
import functools

import jax
import jax.numpy as jnp
from jax import lax
from jax.experimental import pallas as pl
from jax.experimental.pallas import tpu as pltpu
from jax.experimental.pallas import tpu_sc as plsc

EMBED_DIM = 32
BATCH = 16384
N_CLASSES = 1_000_000

_NUM_CORES = 2
_NUM_SUBCORES = 16
_NUM_WORKERS = _NUM_CORES * _NUM_SUBCORES
_B_PER_W = BATCH // _NUM_WORKERS
_SUB = 8
_N_SUB = _B_PER_W // _SUB
_LANES = 128

_mesh = plsc.VectorSubcoreMesh(core_axis_name="c", subcore_axis_name="s")


@functools.partial(
    pl.kernel,
    mesh=_mesh,
    out_type=jax.ShapeDtypeStruct((EMBED_DIM, BATCH), jnp.float32),
    scratch_types=[
        pltpu.VMEM((_B_PER_W + 8,), jnp.int32),
        pltpu.VMEM((2, _SUB, 4, 8, _LANES), jnp.float32),
        pltpu.VMEM((EMBED_DIM, _LANES), jnp.float32),
        pltpu.SemaphoreType.DMA,
        pltpu.SemaphoreType.DMA,
    ],
    compiler_params=pltpu.CompilerParams(needs_layout_passes=False),
)
def _sc_lane_gather(idx_hbm, t3_hbm, out_hbm, idx_v, gbuf, obuf, sem0, sem1):
    wid = lax.axis_index("s") * _NUM_CORES + lax.axis_index("c")
    base = wid * _B_PER_W
    pltpu.sync_copy(idx_hbm.at[pl.ds(base, _B_PER_W)], idx_v.at[pl.ds(0, _B_PER_W)])

    half = lax.iota(jnp.int32, 16) < _SUB
    sems = (sem0, sem1)

    def issue(r, slot):
        cvec = idx_v[pl.ds(r * _SUB, 16)]
        for i in range(_SUB):
            c = cvec[i]
            blk = pl.multiple_of((c >> 7) << 7, _LANES)
            pltpu.async_copy(
                t3_hbm.at[:, :, pl.ds(blk, _LANES)],
                gbuf.at[slot, i],
                sems[slot],
            )

    def drain(slot):
        for i in range(_SUB):
            pltpu.make_async_copy(
                t3_hbm.at[:, :, pl.ds(0, _LANES)],
                gbuf.at[slot, i],
                sems[slot],
            ).wait()

    def extract(r, slot):
        cvec16 = idx_v[pl.ds(r * _SUB, 16)]
        pvec = lax.bitwise_and(cvec16, _LANES - 1)
        svec = jnp.full((16,), slot, jnp.int32)
        ivec = lax.rem(lax.iota(jnp.int32, 16), jnp.int32(_SUB))
        col = lax.rem(r, jnp.int32(16)) * _SUB
        for j in range(EMBED_DIM):
            a = jnp.full((16,), j // 8, jnp.int32)
            b = jnp.full((16,), j % 8, jnp.int32)
            vals = plsc.load_gather(gbuf, [svec, ivec, a, b, pvec], mask=half)
            plsc.store_scatter(
                obuf,
                [jnp.full((16,), j, jnp.int32),
                 col + lax.iota(jnp.int32, 16)],
                vals,
                mask=half,
            )

    def flush(r):
        @pl.when(lax.rem(r, jnp.int32(16)) == 15)
        def _():
            tile = (r // 16) * _LANES
            off = pl.multiple_of(base + tile, _LANES)
            pltpu.sync_copy(obuf, out_hbm.at[:, pl.ds(off, _LANES)])

    issue(0, 0)

    def step(k, carry):
        ra = 2 * k + 1
        issue(ra, 1)
        drain(0)
        extract(ra - 1, 0)
        flush(ra - 1)

        rb = 2 * k + 2

        @pl.when(rb < _N_SUB)
        def _():
            issue(rb, 0)

        drain(1)
        extract(ra, 1)
        flush(ra)
        return carry

    lax.fori_loop(0, _N_SUB // 2, step, 0)


def kernel(c, table):
    t3 = table.T.reshape(4, 8, N_CLASSES)
    out_t = _sc_lane_gather(c.astype(jnp.int32), t3)
    return out_t.T


# speedup vs baseline: 4.6620x; 1.1924x over previous
import functools

import jax
import jax.numpy as jnp
from jax import lax
from jax.experimental import pallas as pl
from jax.experimental.pallas import tpu as pltpu
from jax.experimental.pallas import tpu_sc as plsc

EMBED_DIM = 32
BATCH = 16384
N_CLASSES = 1_000_000

_NUM_CORES = 2
_NUM_SUBCORES = 16
_NUM_WORKERS = _NUM_CORES * _NUM_SUBCORES
_B_PER_W = BATCH // _NUM_WORKERS
_SUB = 8
_N_SUB = _B_PER_W // _SUB
_LANES = 128

_mesh = plsc.VectorSubcoreMesh(core_axis_name="c", subcore_axis_name="s")


@functools.partial(
    pl.kernel,
    mesh=_mesh,
    out_type=jax.ShapeDtypeStruct((EMBED_DIM, BATCH), jnp.float32),
    scratch_types=[
        pltpu.VMEM((_B_PER_W + 8,), jnp.int32),
        pltpu.VMEM((2, _SUB, 4, 8, _LANES), jnp.float32),
        pltpu.VMEM((EMBED_DIM, _LANES), jnp.float32),
        pltpu.SemaphoreType.DMA,
        pltpu.SemaphoreType.DMA,
    ],
    compiler_params=pltpu.CompilerParams(needs_layout_passes=False),
)
def _sc_lane_gather(idx_hbm, t3_hbm, out_hbm, idx_v, gbuf, obuf, sem0, sem1):
    wid = lax.axis_index("s") * _NUM_CORES + lax.axis_index("c")
    base = wid * _B_PER_W
    pltpu.sync_copy(idx_hbm.at[pl.ds(base, _B_PER_W)], idx_v.at[pl.ds(0, _B_PER_W)])

    half = lax.iota(jnp.int32, 16) < _SUB
    sems = (sem0, sem1)

    def issue(r, slot):
        cvec = idx_v[pl.ds(r * _SUB, 16)]
        for i in range(_SUB):
            c = cvec[i]
            blk = pl.multiple_of((c >> 7) << 7, _LANES)
            lo = lax.bitwise_and(c, 127) < 64

            @pl.when(lo)
            def _():
                pltpu.async_copy(
                    t3_hbm.at[:, :, pl.ds(blk, 64)],
                    gbuf.at[slot, i, :, :, pl.ds(0, 64)],
                    sems[slot],
                )

            @pl.when(jnp.logical_not(lo))
            def _():
                pltpu.async_copy(
                    t3_hbm.at[:, :, pl.ds(blk, _LANES)],
                    gbuf.at[slot, i],
                    sems[slot],
                )

    def drain(r, slot):
        cvec = idx_v[pl.ds(r * _SUB, 16)]
        for i in range(_SUB):
            c = cvec[i]
            lo = lax.bitwise_and(c, 127) < 64

            @pl.when(lo)
            def _():
                pltpu.make_async_copy(
                    t3_hbm.at[:, :, pl.ds(0, 64)],
                    gbuf.at[slot, i, :, :, pl.ds(0, 64)],
                    sems[slot],
                ).wait()

            @pl.when(jnp.logical_not(lo))
            def _():
                pltpu.make_async_copy(
                    t3_hbm.at[:, :, pl.ds(0, _LANES)],
                    gbuf.at[slot, i],
                    sems[slot],
                ).wait()

    def extract(r, slot):
        cvec16 = idx_v[pl.ds(r * _SUB, 16)]
        pvec = lax.bitwise_and(cvec16, _LANES - 1)
        svec = jnp.full((16,), slot, jnp.int32)
        ivec = lax.rem(lax.iota(jnp.int32, 16), jnp.int32(_SUB))
        col = lax.rem(r, jnp.int32(16)) * _SUB
        for j in range(EMBED_DIM):
            a = jnp.full((16,), j // 8, jnp.int32)
            b = jnp.full((16,), j % 8, jnp.int32)
            vals = plsc.load_gather(gbuf, [svec, ivec, a, b, pvec], mask=half)
            plsc.store_scatter(
                obuf,
                [jnp.full((16,), j, jnp.int32),
                 col + lax.iota(jnp.int32, 16)],
                vals,
                mask=half,
            )

    def flush(r):
        @pl.when(lax.rem(r, jnp.int32(16)) == 15)
        def _():
            tile = (r // 16) * _LANES
            off = pl.multiple_of(base + tile, _LANES)
            pltpu.sync_copy(obuf, out_hbm.at[:, pl.ds(off, _LANES)])

    issue(0, 0)

    def step(k, carry):
        ra = 2 * k + 1
        issue(ra, 1)
        drain(ra - 1, 0)
        extract(ra - 1, 0)
        flush(ra - 1)

        rb = 2 * k + 2

        @pl.when(rb < _N_SUB)
        def _():
            issue(rb, 0)

        drain(ra, 1)
        extract(ra, 1)
        flush(ra)
        return carry

    lax.fori_loop(0, _N_SUB // 2, step, 0)


def kernel(c, table):
    t3 = table.T.reshape(4, 8, N_CLASSES)
    out_t = _sc_lane_gather(c.astype(jnp.int32), t3)
    return out_t.T
